# baseline (speedup 1.0000x reference)
import functools

import jax
import jax.numpy as jnp
from jax import lax
from jax.experimental import pallas as pl
from jax.experimental.pallas import tpu as pltpu
from jax.experimental.pallas import tpu_sc as plsc

_K = 32
_T = 16
_NNZ = 64
_NC = 2
_NS = 16
_NW = _NC * _NS
_SEC = 2048


def _sc_gather(ut1, it1, uidx, iidx, nrows_u, nrows_i):
    bu = uidx.shape[0] // _NW
    bi = iidx.shape[0] // _NW
    bm = max(bu, bi)
    mesh = plsc.VectorSubcoreMesh(core_axis_name="c", subcore_axis_name="s")

    @functools.partial(
        pl.kernel,
        mesh=mesh,
        compiler_params=pltpu.CompilerParams(use_tc_tiling_on_sc=False),
        out_type=[
            jax.ShapeDtypeStruct((_K, uidx.shape[0]), jnp.float32),
            jax.ShapeDtypeStruct((_K, iidx.shape[0]), jnp.float32),
        ],
        scratch_types=[
            pltpu.VMEM((bm,), jnp.int32),
            pltpu.VMEM((bm,), jnp.int32),
            pltpu.VMEM((_K, bm), jnp.float32),
            pltpu.SemaphoreType.DMA,
        ],
    )
    def gather_kernel(ut_hbm, it_hbm, uq_hbm, iq_hbm, uout_hbm, iout_hbm,
                      idx_v, idxk_v, stage_v, sem):
        wid = lax.axis_index("s") * _NC + lax.axis_index("c")

        def do_stream(tab, nrows, idx_hbm, out_hbm, base, n):
            pltpu.sync_copy(idx_hbm.at[pl.ds(base, n)], idx_v.at[pl.ds(0, n)])

            @pl.loop(0, _K)
            def _(k):
                off = k * nrows

                @pl.loop(0, n, step=16)
                def _(r):
                    idxk_v[pl.ds(r, 16)] = idx_v[pl.ds(r, 16)] + off

                pltpu.async_copy(tab.at[idxk_v.at[pl.ds(0, n)]],
                                 stage_v.at[k, pl.ds(0, n)], sem).wait()

            pltpu.sync_copy(stage_v.at[:, pl.ds(0, n)],
                            out_hbm.at[:, pl.ds(base, n)])

        do_stream(ut_hbm, nrows_u, uq_hbm, uout_hbm, wid * bu, bu)
        do_stream(it_hbm, nrows_i, iq_hbm, iout_hbm, wid * bi, bi)

    return gather_kernel(ut1, it1, uidx, iidx)


def _cbuild_body(wt_ref, rows_ref, vals_ref, ct_ref):
    tag = lax.broadcasted_iota(jnp.int32, (_NNZ, _T), 1)
    S = jnp.where(tag == rows_ref[...], vals_ref[...], jnp.float32(0.0))
    ct_ref[...] = lax.dot_general(
        wt_ref[:, 0:_NNZ], S, (((1,), (0,)), ((), ())),
        preferred_element_type=jnp.float32,
        precision=lax.Precision.HIGHEST)


def _main_body(ut_ref, vpt_ref, vnt_ref, ct_ref, z_ref):
    CT = ct_ref[...]
    dims = (((0,), (0,)), ((), ()))
    mm = functools.partial(lax.dot_general, dimension_numbers=dims,
                           preferred_element_type=jnp.float32,
                           precision=lax.Precision.HIGHEST)
    A = mm(CT, ut_ref[...])
    Bv = mm(CT, vpt_ref[...] - vnt_ref[...])
    z_ref[...] = jnp.sum(A * Bv, axis=0, keepdims=True)


def kernel(samples, neg_item, user_table, item_table, rows, cols, vals):
    B = samples.shape[0]
    nu = user_table.shape[0]
    ni = item_table.shape[0]
    user_idx = samples[:, 0]
    item_idx = jnp.concatenate([
        cols, jnp.zeros((_SEC - _NNZ,), dtype=cols.dtype),
        samples[:, 1], neg_item,
        jnp.zeros((_SEC,), dtype=cols.dtype),
    ])
    NI = item_idx.shape[0]

    ut1 = user_table.T.reshape(-1)
    it1 = item_table.T.reshape(-1)
    ut_t, it_t = _sc_gather(ut1, it1, user_idx, item_idx, nu, ni)

    CT = pl.pallas_call(
        _cbuild_body,
        grid=(1,),
        out_shape=jax.ShapeDtypeStruct((_K, _T), jnp.float32),
        in_specs=[
            pl.BlockSpec((_K, 128), lambda g: (0, 0)),
            pl.BlockSpec((_NNZ, 1), lambda g: (0, 0)),
            pl.BlockSpec((_NNZ, 1), lambda g: (0, 0)),
        ],
        out_specs=pl.BlockSpec((_K, _T), lambda g: (0, 0)),
    )(it_t, rows.reshape(_NNZ, 1), vals.reshape(_NNZ, 1))

    nsec = B // _SEC
    z = pl.pallas_call(
        _main_body,
        grid=(nsec,),
        out_shape=jax.ShapeDtypeStruct((1, B), jnp.float32),
        in_specs=[
            pl.BlockSpec((_K, _SEC), lambda g: (0, g)),
            pl.BlockSpec((_K, _SEC), lambda g: (0, g + 1)),
            pl.BlockSpec((_K, _SEC), lambda g: (0, g + 1 + nsec)),
            pl.BlockSpec((_K, _T), lambda g: (0, 0)),
        ],
        out_specs=pl.BlockSpec((1, _SEC), lambda g: (0, g)),
    )(ut_t, it_t, it_t, CT)
    return z.reshape(B, 1)


# speedup vs baseline: 7.4200x; 7.4200x over previous
import functools

import jax
import jax.numpy as jnp
from jax import lax
from jax.experimental import pallas as pl
from jax.experimental.pallas import tpu as pltpu
from jax.experimental.pallas import tpu_sc as plsc

_K = 32
_T = 16
_NNZ = 64
_NC = 2
_NS = 16
_NW = _NC * _NS
_SEC = 2048
_UCH = 2
_ICH = 4


def _sc_gather(user_table, item_table, uidx, iidx):
    bu = uidx.shape[0] // _NW
    bi = iidx.shape[0] // _NW
    chu = bu // _UCH
    chi = bi // _ICH
    mesh = plsc.VectorSubcoreMesh(core_axis_name="c", subcore_axis_name="s")

    @functools.partial(
        pl.kernel,
        mesh=mesh,
        out_type=[
            jax.ShapeDtypeStruct((uidx.shape[0], _K), jnp.float32),
            jax.ShapeDtypeStruct((iidx.shape[0], _K), jnp.float32),
        ],
        scratch_types=[
            pltpu.VMEM((max(chu, chi),), jnp.int32),
            pltpu.VMEM((max(chu, chi), _K), jnp.float32),
            pltpu.SemaphoreType.DMA,
        ],
    )
    def gather_kernel(ut_hbm, it_hbm, uq_hbm, iq_hbm, uout_hbm, iout_hbm,
                      idx_v, rows_v, sem):
        wid = lax.axis_index("s") * _NC + lax.axis_index("c")

        def do_chunk(tab, idx_hbm, out_hbm, base, n):
            pltpu.sync_copy(idx_hbm.at[pl.ds(base, n)], idx_v.at[pl.ds(0, n)])

            @pl.loop(0, n, step=16)
            def _(r):
                vec = idx_v[pl.ds(r, 16)]
                for l in range(16):
                    pltpu.async_copy(tab.at[pl.ds(vec[l], 1)],
                                     rows_v.at[pl.ds(r + l, 1)], sem)

            pltpu.make_async_copy(tab.at[pl.ds(0, n)],
                                  rows_v.at[pl.ds(0, n)], sem).wait()
            pltpu.sync_copy(rows_v.at[pl.ds(0, n)],
                            out_hbm.at[pl.ds(base, n)])

        for j in range(_UCH):
            do_chunk(ut_hbm, uq_hbm, uout_hbm, wid * bu + j * chu, chu)
        for j in range(_ICH):
            do_chunk(it_hbm, iq_hbm, iout_hbm, wid * bi + j * chi, chi)

    return gather_kernel(user_table, item_table, uidx, iidx)


def _cbuild_body(wraw_ref, rows_ref, vals_ref, c_ref):
    tag = lax.broadcasted_iota(jnp.int32, (_T, _NNZ), 0)
    S = jnp.where(tag == rows_ref[...], vals_ref[...], jnp.float32(0.0))
    c_ref[...] = lax.dot_general(
        S, wraw_ref[...], (((1,), (0,)), ((), ())),
        preferred_element_type=jnp.float32,
        precision=lax.Precision.HIGHEST)


def _main_body(u_ref, vp_ref, vn_ref, c_ref, z_ref):
    C = c_ref[...]
    dims = (((1,), (1,)), ((), ()))
    mm = functools.partial(lax.dot_general, dimension_numbers=dims,
                           preferred_element_type=jnp.float32,
                           precision=lax.Precision.HIGHEST)
    a = mm(u_ref[...], C)
    b = mm(vp_ref[...] - vn_ref[...], C)
    z_ref[...] = jnp.sum(a * b, axis=1, keepdims=True)


def kernel(samples, neg_item, user_table, item_table, rows, cols, vals):
    B = samples.shape[0]
    user_idx = samples[:, 0]
    item_idx = jnp.concatenate([
        cols, jnp.zeros((_SEC - _NNZ,), dtype=cols.dtype),
        samples[:, 1], neg_item,
    ])
    NI = item_idx.shape[0]

    raw_u, raw_i = _sc_gather(user_table, item_table, user_idx, item_idx)

    C = pl.pallas_call(
        _cbuild_body,
        grid=(1,),
        out_shape=jax.ShapeDtypeStruct((_T, _K), jnp.float32),
        in_specs=[
            pl.BlockSpec((_NNZ, _K), lambda g: (0, 0)),
            pl.BlockSpec((1, _NNZ), lambda g: (0, 0)),
            pl.BlockSpec((1, _NNZ), lambda g: (0, 0)),
        ],
        out_specs=pl.BlockSpec((_T, _K), lambda g: (0, 0)),
    )(raw_i, rows.reshape(1, _NNZ), vals.reshape(1, _NNZ))

    nsec = B // _SEC
    z = pl.pallas_call(
        _main_body,
        grid=(nsec,),
        out_shape=jax.ShapeDtypeStruct((B, 1), jnp.float32),
        in_specs=[
            pl.BlockSpec((_SEC, _K), lambda g: (g, 0)),
            pl.BlockSpec((_SEC, _K), lambda g: (g + 1, 0)),
            pl.BlockSpec((_SEC, _K), lambda g: (g + 1 + nsec, 0)),
            pl.BlockSpec((_T, _K), lambda g: (0, 0)),
        ],
        out_specs=pl.BlockSpec((_SEC, 1), lambda g: (g, 0)),
    )(raw_u, raw_i, raw_i, C)
    return z
